# baseline (speedup 1.0000x reference)
import functools

import jax
import jax.numpy as jnp
from jax import lax
from jax.experimental import pallas as pl
from jax.experimental.pallas import tpu as pltpu
from jax.experimental.pallas import tpu_sc as plsc

NUM_RADIAL = 128
NUM_SPH = 8
EMB = 32
OUT_COLS = NUM_SPH * EMB

_NC = 2
_NS = 16
_NW = _NC * _NS
_CHUNK = 128


def _sc_gather(table, idx, t_pad, nc0, nc1):
    mesh = plsc.VectorSubcoreMesh(core_axis_name="c", subcore_axis_name="s")

    @functools.partial(
        pl.kernel,
        mesh=mesh,
        out_type=jax.ShapeDtypeStruct((t_pad, NUM_RADIAL), jnp.float32),
        scratch_types=[
            pltpu.VMEM((_CHUNK,), jnp.int32),
            pltpu.VMEM((_CHUNK,), jnp.int32),
            pltpu.VMEM((_CHUNK, NUM_RADIAL), jnp.float32),
            pltpu.VMEM((_CHUNK, NUM_RADIAL), jnp.float32),
            pltpu.SemaphoreType.DMA,
            pltpu.SemaphoreType.DMA,
            pltpu.SemaphoreType.DMA,
            pltpu.SemaphoreType.DMA,
        ],
    )
    def k(table_hbm, idx_hbm, out_hbm, idx0, idx1, rows0, rows1,
          g0, g1, w0, w1):
        c_ax = lax.axis_index("c")
        s_ax = lax.axis_index("s")
        my_n = jnp.where(c_ax == 0, nc0, nc1)
        base = jnp.where(c_ax == 0, s_ax * nc0, _NS * nc0 + s_ax * nc1)

        def off(c):
            return (base + c) * _CHUNK

        def do_chunk(c, idxb, rowsb, gsem, wsem, drain_first):
            pltpu.sync_copy(idx_hbm.at[pl.ds(off(c), _CHUNK)], idxb)
            if drain_first:
                pltpu.make_async_copy(
                    rowsb, out_hbm.at[pl.ds(off(c), _CHUNK)], wsem).wait()
            pltpu.async_copy(table_hbm.at[idxb], rowsb, gsem).wait()
            pltpu.async_copy(rowsb, out_hbm.at[pl.ds(off(c), _CHUNK)], wsem)

        do_chunk(0, idx0, rows0, g0, w0, False)
        do_chunk(1, idx1, rows1, g1, w1, False)

        def body(j, carry):
            do_chunk(2 * j, idx0, rows0, g0, w0, True)
            do_chunk(2 * j + 1, idx1, rows1, g1, w1, True)
            return carry

        lax.fori_loop(1, my_n // 2, body, 0, unroll=False)
        pltpu.make_async_copy(
            rows0, out_hbm.at[pl.ds(off(my_n - 2), _CHUNK)], w0).wait()
        pltpu.make_async_copy(
            rows1, out_hbm.at[pl.ds(off(my_n - 1), _CHUNK)], w1).wait()

    return k(table, idx)


def _tc_contract(g, sph, wt, t, tile):
    def body(g_ref, s_ref, w_ref, o_ref):
        h = jnp.dot(g_ref[...], w_ref[...], preferred_element_type=jnp.float32)
        s = s_ref[...]
        acc = h[:, :EMB] * s[:, 0:1]
        for b in range(1, NUM_SPH):
            acc = acc + h[:, b * EMB:(b + 1) * EMB] * s[:, b:b + 1]
        o_ref[...] = acc

    return pl.pallas_call(
        body,
        grid=(t // tile,),
        in_specs=[
            pl.BlockSpec((tile, NUM_RADIAL), lambda i: (i, 0)),
            pl.BlockSpec((tile, NUM_SPH), lambda i: (i, 0)),
            pl.BlockSpec((NUM_RADIAL, OUT_COLS), lambda i: (0, 0)),
        ],
        out_specs=pl.BlockSpec((tile, EMB), lambda i: (i, 0)),
        out_shape=jax.ShapeDtypeStruct((t, EMB), jnp.float32),
    )(g, sph, wt)


def kernel(rbf, sph, idx_sph, weight):
    t = idx_sph.shape[0]
    tile = 3000
    nchunks = -(-t // (_NW * _CHUNK))
    nchunks += nchunks % 2
    t_pad = _NW * nchunks * _CHUNK
    nc0 = max(4, (2 * nchunks * 13 // 20) // 2 * 2)
    nc1 = 2 * nchunks - nc0

    idx_pad = jnp.zeros((t_pad,), jnp.int32).at[:t].set(idx_sph)

    g = _sc_gather(rbf, idx_pad, t_pad, nc0, nc1)

    wt = jnp.transpose(
        weight.reshape(NUM_RADIAL, EMB, NUM_SPH), (0, 2, 1)
    ).reshape(NUM_RADIAL, OUT_COLS)

    return _tc_contract(g, sph, wt, t, tile)
